# baseline (speedup 1.0000x reference)
import jax
import jax.numpy as jnp
from jax import lax
from jax.experimental import pallas as pl
from jax.experimental.pallas import tpu as pltpu
from jax.experimental.pallas import tpu_sc as plsc

N = 128
A = 512
C = 1000
L = 16
NG = 4
NR = 8
RH = 128
GW = A // NG
KV = GW // L

MOMENTUM = 0.8


def _body(feat_hbm, lab_hbm, cov_hbm, mean_hbm, amt_hbm,
          cov_out, mean_out, amt_out,
          lab_v, feat_v, cov_blk, mean_blk, amt_w, cnt_w,
          acc_sum, acc_sq, amt_new_w,
          sem_lab, sem_feat, sem_amt, sem_cov, sem_mean):
    nc = 2
    wid = lax.axis_index("s") * nc + lax.axis_index("c")
    r = wid // NG
    g = wid - r * NG
    rb = jnp.minimum(r * RH, C - RH)
    cb = g * GW

    c_lab = pltpu.async_copy(lab_hbm, lab_v, sem_lab)
    c_feat = pltpu.async_copy(feat_hbm.at[:, pl.ds(cb, GW)], feat_v, sem_feat)
    c_amt = pltpu.async_copy(amt_hbm.at[pl.ds(rb, RH)],
                             amt_w.at[pl.ds(0, RH)], sem_amt)
    c_cov = pltpu.async_copy(cov_hbm.at[pl.ds(rb, RH), pl.ds(cb, GW)],
                             cov_blk, sem_cov)
    c_mean = pltpu.async_copy(mean_hbm.at[pl.ds(rb, RH), pl.ds(cb, GW)],
                              mean_blk, sem_mean)

    zeros = jnp.zeros((L,), jnp.float32)
    ones = jnp.ones((L,), jnp.float32)
    zeros_i = jnp.zeros((L,), jnp.int32)
    lanes = lax.iota(jnp.int32, L)

    @plsc.parallel_loop(0, (RH + L) // L, unroll=2)
    def _(i):
        cnt_w[pl.ds(i * L, L)] = zeros

    @plsc.parallel_loop(0, RH, unroll=4)
    def _(row):
        for k in range(KV):
            s = pl.ds(k * L, L)
            acc_sum[row, s] = zeros
            acc_sq[row, s] = zeros

    c_lab.wait()
    c_feat.wait()

    @plsc.parallel_loop(0, N // L, unroll=4)
    def _(i):
        lab16 = lab_v[pl.ds(i * L, L)]
        lr16 = lab16 - rb
        m = jnp.logical_and(lr16 >= 0, lr16 < RH)
        idx = jnp.where(m, lr16, 0)
        plsc.addupdate_scatter(cnt_w, [idx], ones, mask=m)
        for j in range(L):
            row16 = lr16[j] + zeros_i
            msk = jnp.logical_and(row16 >= 0, row16 < RH)
            rowc = jnp.where(msk, row16, 0)
            n = i * L + j
            for k in range(KV):
                s = pl.ds(k * L, L)
                f = feat_v[n, s]
                col = k * L + lanes
                plsc.addupdate_scatter(acc_sum, [rowc, col], f, mask=msk)
                plsc.addupdate_scatter(acc_sq, [rowc, col], f * f, mask=msk)

    c_amt.wait()

    @pl.when(g == 0)
    def _():
        @plsc.parallel_loop(0, RH // L)
        def _(k):
            s = pl.ds(k * L, L)
            amt_new_w[s] = amt_w[s] + cnt_w[s]

        pltpu.sync_copy(amt_new_w, amt_out.at[pl.ds(rb, RH)])

    c_cov.wait()
    c_mean.wait()

    H = RH // 2

    def update(lr):
        cnt = cnt_w[pl.ds(lr, L)][0]

        @pl.when(cnt > 0.0)
        def _():
            cntv = cnt + zeros
            amtv = amt_w[pl.ds(lr, L)][0] + zeros
            w = jnp.maximum(cntv / (cntv + amtv), 1.0 - MOMENTUM)
            rc = ones / cntv
            omw = 1.0 - w
            for k in range(KV):
                s = pl.ds(k * L, L)
                ave = acc_sum[lr, s] * rc
                var = acc_sq[lr, s] * rc - ave * ave
                mn = mean_blk[lr, s]
                dm = mn - ave
                cov_blk[lr, s] = (cov_blk[lr, s] * omw + var * w
                                  + w * omw * dm * dm)
                mean_blk[lr, s] = mn * omw + ave * w

    plsc.parallel_loop(0, H, unroll=2)(update)

    c_cov_o1 = pltpu.async_copy(
        cov_blk.at[pl.ds(0, H)],
        cov_out.at[pl.ds(rb, H), pl.ds(cb, GW)], sem_cov)
    c_mean_o1 = pltpu.async_copy(
        mean_blk.at[pl.ds(0, H)],
        mean_out.at[pl.ds(rb, H), pl.ds(cb, GW)], sem_mean)

    plsc.parallel_loop(H, RH, unroll=2)(update)

    c_cov_o2 = pltpu.async_copy(
        cov_blk.at[pl.ds(H, H)],
        cov_out.at[pl.ds(rb + H, H), pl.ds(cb, GW)], sem_cov)
    c_mean_o2 = pltpu.async_copy(
        mean_blk.at[pl.ds(H, H)],
        mean_out.at[pl.ds(rb + H, H), pl.ds(cb, GW)], sem_mean)
    c_cov_o1.wait()
    c_mean_o1.wait()
    c_cov_o2.wait()
    c_mean_o2.wait()


_sc_call = pl.kernel(
    _body,
    out_type=(
        jax.ShapeDtypeStruct((C, A), jnp.float32),
        jax.ShapeDtypeStruct((C, A), jnp.float32),
        jax.ShapeDtypeStruct((C,), jnp.float32),
    ),
    mesh=plsc.VectorSubcoreMesh(core_axis_name="c", subcore_axis_name="s"),
    compiler_params=pltpu.CompilerParams(use_tc_tiling_on_sc=True,
                                         needs_layout_passes=False),
    scratch_types=[
        pltpu.VMEM((N,), jnp.int32),
        pltpu.VMEM((N, GW), jnp.float32),
        pltpu.VMEM((RH, GW), jnp.float32),
        pltpu.VMEM((RH, GW), jnp.float32),
        pltpu.VMEM((RH + L,), jnp.float32),
        pltpu.VMEM((RH + L,), jnp.float32),
        pltpu.VMEM((RH, GW), jnp.float32),
        pltpu.VMEM((RH, GW), jnp.float32),
        pltpu.VMEM((RH,), jnp.float32),
        pltpu.SemaphoreType.DMA,
        pltpu.SemaphoreType.DMA,
        pltpu.SemaphoreType.DMA,
        pltpu.SemaphoreType.DMA,
        pltpu.SemaphoreType.DMA,
    ],
)


@jax.jit
def kernel(features, labels, covariance, mean, amount):
    return _sc_call(features, labels, covariance, mean, amount)


# speedup vs baseline: 1.1071x; 1.1071x over previous
import jax
import jax.numpy as jnp
from jax import lax
from jax.experimental import pallas as pl
from jax.experimental.pallas import tpu as pltpu
from jax.experimental.pallas import tpu_sc as plsc

N = 128
A = 512
C = 1000
L = 16
NG = 4
NR = 8
RH = 128
GW = A // NG
KV = GW // L

MOMENTUM = 0.8


def _body(feat_hbm, lab_hbm, cov_hbm, mean_hbm, amt_hbm,
          cov_out, mean_out, amt_out,
          lab_v, feat_v, cov_blk, mean_blk, amt_w, cnt_w,
          acc_sum, acc_sq, amt_new_w,
          sem_lab, sem_feat, sem_amt, sem_cov, sem_mean):
    nc = 2
    wid = lax.axis_index("s") * nc + lax.axis_index("c")
    r = wid // NG
    g = wid - r * NG
    rb = jnp.minimum(r * RH, C - RH)
    cb = g * GW

    c_lab = pltpu.async_copy(lab_hbm, lab_v, sem_lab)
    c_feat = pltpu.async_copy(feat_hbm.at[:, pl.ds(cb, GW)], feat_v, sem_feat)
    c_amt = pltpu.async_copy(amt_hbm.at[pl.ds(rb, RH)],
                             amt_w.at[pl.ds(0, RH)], sem_amt)
    c_cov = pltpu.async_copy(cov_hbm.at[pl.ds(rb, RH), pl.ds(cb, GW)],
                             cov_blk, sem_cov)
    c_mean = pltpu.async_copy(mean_hbm.at[pl.ds(rb, RH), pl.ds(cb, GW)],
                              mean_blk, sem_mean)

    zeros = jnp.zeros((L,), jnp.float32)
    ones = jnp.ones((L,), jnp.float32)
    zeros_i = jnp.zeros((L,), jnp.int32)
    lanes = lax.iota(jnp.int32, L)

    @plsc.parallel_loop(0, (RH + L) // L, unroll=2)
    def _(i):
        cnt_w[pl.ds(i * L, L)] = zeros

    @plsc.parallel_loop(0, RH, unroll=4)
    def _(row):
        for k in range(KV):
            s = pl.ds(k * L, L)
            acc_sum[row, s] = zeros
            acc_sq[row, s] = zeros

    c_lab.wait()
    c_feat.wait()

    @plsc.parallel_loop(0, N // L, unroll=2)
    def _(i):
        lab16 = lab_v[pl.ds(i * L, L)]
        lr16 = lab16 - rb
        m = jnp.logical_and(lr16 >= 0, lr16 < RH)
        idx = jnp.where(m, lr16, 0)
        plsc.addupdate_scatter(cnt_w, [idx], ones, mask=m)
        for j in range(L):
            row16 = lr16[j] + zeros_i
            msk = jnp.logical_and(row16 >= 0, row16 < RH)
            rowc = jnp.where(msk, row16, 0)
            n = i * L + j
            for k in range(KV):
                s = pl.ds(k * L, L)
                f = feat_v[n, s]
                col = k * L + lanes
                plsc.addupdate_scatter(acc_sum, [rowc, col], f, mask=msk)
                plsc.addupdate_scatter(acc_sq, [rowc, col], f * f, mask=msk)

    c_amt.wait()

    @pl.when(g == 0)
    def _():
        @plsc.parallel_loop(0, RH // L)
        def _(k):
            s = pl.ds(k * L, L)
            amt_new_w[s] = amt_w[s] + cnt_w[s]

        pltpu.sync_copy(amt_new_w, amt_out.at[pl.ds(rb, RH)])

    c_cov.wait()
    c_mean.wait()

    H = RH // 2

    def update(lr):
        cnt = cnt_w[pl.ds(lr, L)][0]

        @pl.when(cnt > 0.0)
        def _():
            cntv = cnt + zeros
            amtv = amt_w[pl.ds(lr, L)][0] + zeros
            w = jnp.maximum(cntv / (cntv + amtv), 1.0 - MOMENTUM)
            rc = ones / cntv
            omw = 1.0 - w
            for k in range(KV):
                s = pl.ds(k * L, L)
                ave = acc_sum[lr, s] * rc
                var = acc_sq[lr, s] * rc - ave * ave
                mn = mean_blk[lr, s]
                dm = mn - ave
                cov_blk[lr, s] = (cov_blk[lr, s] * omw + var * w
                                  + w * omw * dm * dm)
                mean_blk[lr, s] = mn * omw + ave * w

    plsc.parallel_loop(0, H, unroll=2)(update)

    c_cov_o1 = pltpu.async_copy(
        cov_blk.at[pl.ds(0, H)],
        cov_out.at[pl.ds(rb, H), pl.ds(cb, GW)], sem_cov)
    c_mean_o1 = pltpu.async_copy(
        mean_blk.at[pl.ds(0, H)],
        mean_out.at[pl.ds(rb, H), pl.ds(cb, GW)], sem_mean)

    plsc.parallel_loop(H, RH, unroll=2)(update)

    c_cov_o2 = pltpu.async_copy(
        cov_blk.at[pl.ds(H, H)],
        cov_out.at[pl.ds(rb + H, H), pl.ds(cb, GW)], sem_cov)
    c_mean_o2 = pltpu.async_copy(
        mean_blk.at[pl.ds(H, H)],
        mean_out.at[pl.ds(rb + H, H), pl.ds(cb, GW)], sem_mean)
    c_cov_o1.wait()
    c_mean_o1.wait()
    c_cov_o2.wait()
    c_mean_o2.wait()


_sc_call = pl.kernel(
    _body,
    out_type=(
        jax.ShapeDtypeStruct((C, A), jnp.float32),
        jax.ShapeDtypeStruct((C, A), jnp.float32),
        jax.ShapeDtypeStruct((C,), jnp.float32),
    ),
    mesh=plsc.VectorSubcoreMesh(core_axis_name="c", subcore_axis_name="s"),
    compiler_params=pltpu.CompilerParams(use_tc_tiling_on_sc=True,
                                         needs_layout_passes=False),
    scratch_types=[
        pltpu.VMEM((N,), jnp.int32),
        pltpu.VMEM((N, GW), jnp.float32),
        pltpu.VMEM((RH, GW), jnp.float32),
        pltpu.VMEM((RH, GW), jnp.float32),
        pltpu.VMEM((RH + L,), jnp.float32),
        pltpu.VMEM((RH + L,), jnp.float32),
        pltpu.VMEM((RH, GW), jnp.float32),
        pltpu.VMEM((RH, GW), jnp.float32),
        pltpu.VMEM((RH,), jnp.float32),
        pltpu.SemaphoreType.DMA,
        pltpu.SemaphoreType.DMA,
        pltpu.SemaphoreType.DMA,
        pltpu.SemaphoreType.DMA,
        pltpu.SemaphoreType.DMA,
    ],
)


@jax.jit
def kernel(features, labels, covariance, mean, amount):
    return _sc_call(features, labels, covariance, mean, amount)
